# baseline (speedup 1.0000x reference)
import functools
import jax
import jax.numpy as jnp
from jax import lax
from jax.experimental import pallas as pl
from jax.experimental.pallas import tpu as pltpu


def _chamfer_body(src_ref, tgtT_ref, out_ref):
    b = pl.program_id(0)
    nb = pl.num_programs(0)
    src = src_ref[0]
    tgtT = tgtT_ref[0]

    sq_s = jnp.sum(src * src, axis=1, keepdims=True)
    sq_t = jnp.sum(tgtT * tgtT, axis=0, keepdims=True)

    lane = lax.broadcasted_iota(jnp.int32, src.shape, 1)
    row = lax.broadcasted_iota(jnp.int32, tgtT.shape, 0)

    src_aug = jnp.where(lane < 3, -2.0 * src, jnp.where(lane == 3, 1.0, 0.0))
    tgt_aug = jnp.where(row == 3, sq_t, tgtT)
    e = jnp.dot(src_aug, tgt_aug, preferred_element_type=jnp.float32)

    src_aug2 = jnp.where(lane < 3, -2.0 * src, jnp.where(lane == 3, sq_s, 0.0))
    tgt_aug2 = jnp.where(row == 3, 1.0, tgtT)
    f = jnp.dot(src_aug2, tgt_aug2, preferred_element_type=jnp.float32)

    rowmin = jnp.min(e, axis=1, keepdims=True)
    colmin = jnp.min(f, axis=0, keepdims=True)

    d_s2t = jnp.maximum(rowmin + sq_s, 0.0)
    d_t2s = jnp.maximum(colmin + sq_t, 0.0)

    n = src.shape[0]
    m = tgtT.shape[1]
    batch_val = jnp.sum(d_s2t) / n + jnp.sum(d_t2s) / m

    @pl.when(b == 0)
    def _():
        out_ref[0, 0] = 0.0

    out_ref[0, 0] += batch_val / nb


@jax.jit
def kernel(src_points, tgt_points):
    B, N, D = src_points.shape
    M = tgt_points.shape[1]
    src_pad = jnp.pad(src_points, ((0, 0), (0, 0), (0, 8 - D)))
    tgtT_pad = jnp.pad(
        jnp.transpose(tgt_points, (0, 2, 1)), ((0, 0), (0, 8 - D), (0, 0))
    )

    out = pl.pallas_call(
        _chamfer_body,
        grid=(B,),
        in_specs=[
            pl.BlockSpec((1, N, 8), lambda b: (b, 0, 0)),
            pl.BlockSpec((1, 8, M), lambda b: (b, 0, 0)),
        ],
        out_specs=pl.BlockSpec((1, 1), lambda b: (0, 0), memory_space=pltpu.SMEM),
        out_shape=jax.ShapeDtypeStruct((1, 1), jnp.float32),
    )(src_pad, tgtT_pad)
    return out[0, 0]


# speedup vs baseline: 1.4360x; 1.4360x over previous
import functools
import jax
import jax.numpy as jnp
from jax import lax
from jax.experimental import pallas as pl
from jax.experimental.pallas import tpu as pltpu


def _chamfer_body(src_ref, tgtT_ref, out_ref):
    b = pl.program_id(0)
    nb = pl.num_programs(0)
    src = src_ref[0]
    tgtT = tgtT_ref[0]

    sq_s = jnp.sum(src * src, axis=1, keepdims=True)
    sq_t = jnp.sum(tgtT * tgtT, axis=0, keepdims=True)

    lane = lax.broadcasted_iota(jnp.int32, src.shape, 1)
    row = lax.broadcasted_iota(jnp.int32, tgtT.shape, 0)

    src_aug = jnp.where(
        lane < 3, -2.0 * src, jnp.where(lane == 3, 1.0, jnp.where(lane == 4, sq_s, 0.0))
    )
    tgt_aug = jnp.where(row == 3, sq_t, jnp.where(row == 4, 1.0, tgtT))
    d2 = jnp.dot(src_aug, tgt_aug, preferred_element_type=jnp.float32)

    rowmin = jnp.min(d2, axis=1, keepdims=True)
    colmin = jnp.min(d2, axis=0, keepdims=True)

    d_s2t = jnp.maximum(rowmin, 0.0)
    d_t2s = jnp.maximum(colmin, 0.0)

    n = src.shape[0]
    m = tgtT.shape[1]
    batch_val = jnp.sum(d_s2t) / n + jnp.sum(d_t2s) / m

    @pl.when(b == 0)
    def _():
        out_ref[0, 0] = 0.0

    out_ref[0, 0] += batch_val / nb


@jax.jit
def kernel(src_points, tgt_points):
    B, N, D = src_points.shape
    M = tgt_points.shape[1]
    src_pad = jnp.pad(src_points, ((0, 0), (0, 0), (0, 8 - D)))
    tgtT_pad = jnp.pad(
        jnp.transpose(tgt_points, (0, 2, 1)), ((0, 0), (0, 8 - D), (0, 0))
    )

    out = pl.pallas_call(
        _chamfer_body,
        grid=(B,),
        in_specs=[
            pl.BlockSpec((1, N, 8), lambda b: (b, 0, 0)),
            pl.BlockSpec((1, 8, M), lambda b: (b, 0, 0)),
        ],
        out_specs=pl.BlockSpec((1, 1), lambda b: (0, 0), memory_space=pltpu.SMEM),
        out_shape=jax.ShapeDtypeStruct((1, 1), jnp.float32),
    )(src_pad, tgtT_pad)
    return out[0, 0]
